# baseline (speedup 1.0000x reference)
import functools

import jax
import jax.numpy as jnp
from jax import lax
from jax.experimental import pallas as pl
from jax.experimental.pallas import tpu as pltpu
from jax.experimental.pallas import tpu_sc as plsc

NM = 1000000
B = 16384
TWO_B = 2 * B
D = 20

_info = plsc.get_sparse_core_info()
_NC, _NS, _NL = _info.num_cores, _info.num_subcores, _info.num_lanes
_NW = _NC * _NS
_PAIR_W = B // _NW

_sc_mesh = plsc.VectorSubcoreMesh(core_axis_name="c", subcore_axis_name="s")


@functools.partial(
    pl.kernel,
    mesh=_sc_mesh,
    compiler_params=pltpu.CompilerParams(needs_layout_passes=False),
    out_type=(
        jax.ShapeDtypeStruct((B, 2 * D), jnp.float32),
        jax.ShapeDtypeStruct((B, 2 * D), jnp.float32),
        jax.ShapeDtypeStruct((B, 2), jnp.float32),
        jax.ShapeDtypeStruct((B, 2), jnp.float32),
    ),
    scratch_types=[
        pltpu.VMEM((2 * _PAIR_W,), jnp.int32),
        pltpu.SemaphoreType.DMA,
        pltpu.SemaphoreType.DMA,
        pltpu.SemaphoreType.DMA,
        pltpu.SemaphoreType.DMA,
    ],
)
def _sc_gather(idx_hbm, bias_hbm, ent_hbm, eu_out, ei_out, bu_out, bi_out,
               idx_v, sem, sem2, sem3, sem4):
    wid = lax.axis_index("s") * _NC + lax.axis_index("c")
    pbase = wid * _PAIR_W
    pltpu.sync_copy(idx_hbm.at[pl.ds(2 * pbase, 2 * _PAIR_W)], idx_v)
    lanes = lax.iota(jnp.int32, _NL)

    def body(p, carry):
        j = 2 * p
        vec = idx_v[pl.ds((j // _NL) * _NL, _NL)]
        u = jnp.max(jnp.where(lanes == j % _NL, vec, 0))
        v = jnp.max(jnp.where(lanes == j % _NL + 1, vec, 0))
        gp = pbase + p
        pltpu.async_copy(ent_hbm.at[pl.ds(u, 1)], eu_out.at[pl.ds(gp, 1)], sem)
        pltpu.async_copy(ent_hbm.at[pl.ds(v, 1)], ei_out.at[pl.ds(gp, 1)], sem2)
        pltpu.async_copy(bias_hbm.at[pl.ds(u, 1)], bu_out.at[pl.ds(gp, 1)], sem3)
        pltpu.async_copy(bias_hbm.at[pl.ds(v, 1)], bi_out.at[pl.ds(gp, 1)], sem4)
        return carry

    lax.fori_loop(0, _PAIR_W, body, 0)
    pltpu.make_async_copy(ent_hbm.at[pl.ds(0, _PAIR_W)],
                          eu_out.at[pl.ds(pbase, _PAIR_W)], sem).wait()
    pltpu.make_async_copy(ent_hbm.at[pl.ds(0, _PAIR_W)],
                          ei_out.at[pl.ds(pbase, _PAIR_W)], sem2).wait()
    pltpu.make_async_copy(bias_hbm.at[pl.ds(0, _PAIR_W)],
                          bu_out.at[pl.ds(pbase, _PAIR_W)], sem3).wait()
    pltpu.make_async_copy(bias_hbm.at[pl.ds(0, _PAIR_W)],
                          bi_out.at[pl.ds(pbase, _PAIR_W)], sem4).wait()


def _tc_body(eu_ref, ei_ref, bu_ref, bi_ref, epsb_ref, epse_ref, alpha_ref,
             gb_ref, pred_ref, klb_ref, kle_ref, std_ref):
    eu = eu_ref[...]
    ei = ei_ref[...]
    mu_u, rho_u = eu[:, 0:D], eu[:, D:2 * D]
    mu_i, rho_i = ei[:, 0:D], ei[:, D:2 * D]
    sc_u = jax.nn.softplus(rho_u)
    sc_i = jax.nn.softplus(rho_i)
    eps = epse_ref[...]
    s_u = mu_u + sc_u * eps[:, 0:D]
    s_i = mu_i + sc_i * eps[:, D:2 * D]
    emb = jnp.sum(s_u * s_i, axis=1)

    bu = bu_ref[...]
    bi = bi_ref[...]
    mu_bu, rho_bu = bu[:, 0], bu[:, 1]
    mu_bi, rho_bi = bi[:, 0], bi[:, 1]
    sb_u = jax.nn.softplus(rho_bu)
    sb_i = jax.nn.softplus(rho_bi)
    epsb = epsb_ref[...]
    pred = (gb_ref[0] + (mu_bu + sb_u * epsb[:, 0])
            + (mu_bi + sb_i * epsb[:, 1]) + emb)
    pred_ref[...] = pred[:, None]

    klb_u = -jnp.log(sb_u) + (sb_u * sb_u + mu_bu * mu_bu) * 0.5 - 0.5
    klb_i = -jnp.log(sb_i) + (sb_i * sb_i + mu_bi * mu_bi) * 0.5 - 0.5
    klb_ref[...] = jnp.stack([klb_u, klb_i], axis=1)

    kle_u = (0.5 * (jnp.sum(sc_u * sc_u + mu_u * mu_u, axis=1) - D)
             - jnp.sum(jnp.log(sc_u), axis=1))
    kle_i = (0.5 * (jnp.sum(sc_i * sc_i + mu_i * mu_i, axis=1) - D)
             - jnp.sum(jnp.log(sc_i), axis=1))
    kle_ref[...] = jnp.stack([kle_u, kle_i], axis=1)

    @pl.when(pl.program_id(0) == 0)
    def _():
        std_ref[...] = jnp.full((1, 1), jnp.sqrt(1.0 / jax.nn.softplus(alpha_ref[0])),
                                dtype=jnp.float32)


_BM = 2048


def _tc_compute(eu, ei, bu, bi, epsb2, epse2, alpha, gb):
    grid = (B // _BM,)
    return pl.pallas_call(
        _tc_body,
        grid=grid,
        in_specs=[
            pl.BlockSpec((_BM, 2 * D), lambda i: (i, 0)),
            pl.BlockSpec((_BM, 2 * D), lambda i: (i, 0)),
            pl.BlockSpec((_BM, 2), lambda i: (i, 0)),
            pl.BlockSpec((_BM, 2), lambda i: (i, 0)),
            pl.BlockSpec((_BM, 2), lambda i: (i, 0)),
            pl.BlockSpec((_BM, 2 * D), lambda i: (i, 0)),
            pl.BlockSpec(memory_space=pltpu.SMEM),
            pl.BlockSpec(memory_space=pltpu.SMEM),
        ],
        out_specs=[
            pl.BlockSpec((_BM, 1), lambda i: (i, 0)),
            pl.BlockSpec((_BM, 2), lambda i: (i, 0)),
            pl.BlockSpec((_BM, 2), lambda i: (i, 0)),
            pl.BlockSpec((1, 1), lambda i: (0, 0)),
        ],
        out_shape=[
            jax.ShapeDtypeStruct((B, 1), jnp.float32),
            jax.ShapeDtypeStruct((B, 2), jnp.float32),
            jax.ShapeDtypeStruct((B, 2), jnp.float32),
            jax.ShapeDtypeStruct((1, 1), jnp.float32),
        ],
    )(eu, ei, bu, bi, epsb2, epse2, alpha, gb)


def kernel(x, bias_table, entity_table, alpha, global_bias):
    idx = x.astype(jnp.int32).reshape(TWO_B)
    eu, ei, bu, bi = _sc_gather(idx, bias_table, entity_table)
    eps_b = jax.random.normal(jax.random.key(1), (1, TWO_B),
                              dtype=jnp.float32).reshape(B, 2)
    eps_e = jax.random.normal(jax.random.key(2), (1, TWO_B, D),
                              dtype=jnp.float32).reshape(B, 2 * D)
    pred2, klb2, kle2, std2 = _tc_compute(eu, ei, bu, bi, eps_b, eps_e,
                                          alpha, global_bias)
    return (pred2.reshape(B), std2.reshape(1),
            klb2.reshape(TWO_B), kle2.reshape(TWO_B))


# speedup vs baseline: 1.7271x; 1.7271x over previous
import functools

import jax
import jax.numpy as jnp
from jax import lax
from jax.experimental import pallas as pl
from jax.experimental.pallas import tpu as pltpu
from jax.experimental.pallas import tpu_sc as plsc

NM = 1000000
B = 16384
TWO_B = 2 * B
D = 20

_info = plsc.get_sparse_core_info()
_NC, _NS, _NL = _info.num_cores, _info.num_subcores, _info.num_lanes
_NW = _NC * _NS
_PAIR_W = B // _NW

_sc_mesh = plsc.VectorSubcoreMesh(core_axis_name="c", subcore_axis_name="s")


@functools.partial(
    pl.kernel,
    mesh=_sc_mesh,
    compiler_params=pltpu.CompilerParams(needs_layout_passes=False),
    out_type=(
        jax.ShapeDtypeStruct((B, 2 * D), jnp.float32),
        jax.ShapeDtypeStruct((B, 2 * D), jnp.float32),
    ),
    scratch_types=[
        pltpu.VMEM((2 * _PAIR_W,), jnp.int32),
        pltpu.SemaphoreType.DMA,
        pltpu.SemaphoreType.DMA,
    ],
)
def _sc_gather_entity(idx_hbm, ent_hbm, eu_out, ei_out, idx_v, sem, sem2):
    wid = lax.axis_index("s") * _NC + lax.axis_index("c")
    pbase = wid * _PAIR_W
    pltpu.sync_copy(idx_hbm.at[pl.ds(2 * pbase, 2 * _PAIR_W)], idx_v)
    lanes = lax.iota(jnp.int32, _NL)

    def body(p, carry):
        j = 2 * p
        vec = idx_v[pl.ds((j // _NL) * _NL, _NL)]
        u = jnp.max(jnp.where(lanes == j % _NL, vec, 0))
        v = jnp.max(jnp.where(lanes == j % _NL + 1, vec, 0))
        gp = pbase + p
        pltpu.async_copy(ent_hbm.at[pl.ds(u, 1)], eu_out.at[pl.ds(gp, 1)], sem)
        pltpu.async_copy(ent_hbm.at[pl.ds(v, 1)], ei_out.at[pl.ds(gp, 1)], sem2)
        return carry

    lax.fori_loop(0, _PAIR_W, body, 0)
    pltpu.make_async_copy(ent_hbm.at[pl.ds(0, _PAIR_W)],
                          eu_out.at[pl.ds(pbase, _PAIR_W)], sem).wait()
    pltpu.make_async_copy(ent_hbm.at[pl.ds(0, _PAIR_W)],
                          ei_out.at[pl.ds(pbase, _PAIR_W)], sem2).wait()


_PER_W = TWO_B // _NW
_CHUNK = 128
_NCHUNK = _PER_W // _CHUNK


@functools.partial(
    pl.kernel,
    mesh=_sc_mesh,
    compiler_params=pltpu.CompilerParams(use_tc_tiling_on_sc=False),
    out_type=(
        jax.ShapeDtypeStruct((TWO_B,), jnp.float32),
        jax.ShapeDtypeStruct((TWO_B,), jnp.float32),
    ),
    scratch_types=[
        pltpu.VMEM((_PER_W,), jnp.int32),
        pltpu.VMEM((_PER_W,), jnp.float32),
        pltpu.VMEM((_PER_W,), jnp.float32),
        pltpu.SemaphoreType.DMA,
    ],
)
def _sc_gather_bias(idx_hbm, b0_hbm, b1_hbm, m_out, r_out,
                    idx_v, m_v, r_v, sem):
    wid = lax.axis_index("s") * _NC + lax.axis_index("c")
    base = wid * _PER_W
    pltpu.sync_copy(idx_hbm.at[pl.ds(base, _PER_W)], idx_v)
    copies = []
    for j in range(_NCHUNK):
        sl = pl.ds(j * _CHUNK, _CHUNK)
        copies.append(pltpu.async_copy(b0_hbm.at[idx_v.at[sl]], m_v.at[sl], sem))
        copies.append(pltpu.async_copy(b1_hbm.at[idx_v.at[sl]], r_v.at[sl], sem))
    for c in copies:
        c.wait()
    pltpu.sync_copy(m_v, m_out.at[pl.ds(base, _PER_W)])
    pltpu.sync_copy(r_v, r_out.at[pl.ds(base, _PER_W)])


def _tc_body(eu_ref, ei_ref, m_ref, r_ref, epsb_ref, epse_ref, alpha_ref,
             gb_ref, pred_ref, klb_ref, kle_ref, std_ref):
    eu = eu_ref[...]
    ei = ei_ref[...]
    mu_u, rho_u = eu[:, 0:D], eu[:, D:2 * D]
    mu_i, rho_i = ei[:, 0:D], ei[:, D:2 * D]
    sc_u = jax.nn.softplus(rho_u)
    sc_i = jax.nn.softplus(rho_i)
    eps = epse_ref[...]
    s_u = mu_u + sc_u * eps[:, 0:D]
    s_i = mu_i + sc_i * eps[:, D:2 * D]
    emb = jnp.sum(s_u * s_i, axis=1)

    mb = m_ref[...]
    rb = r_ref[...]
    mu_bu, mu_bi = mb[:, 0], mb[:, 1]
    rho_bu, rho_bi = rb[:, 0], rb[:, 1]
    sb_u = jax.nn.softplus(rho_bu)
    sb_i = jax.nn.softplus(rho_bi)
    epsb = epsb_ref[...]
    pred = (gb_ref[0] + (mu_bu + sb_u * epsb[:, 0])
            + (mu_bi + sb_i * epsb[:, 1]) + emb)
    pred_ref[...] = pred[:, None]

    klb_u = -jnp.log(sb_u) + (sb_u * sb_u + mu_bu * mu_bu) * 0.5 - 0.5
    klb_i = -jnp.log(sb_i) + (sb_i * sb_i + mu_bi * mu_bi) * 0.5 - 0.5
    klb_ref[...] = jnp.stack([klb_u, klb_i], axis=1)

    kle_u = (0.5 * (jnp.sum(sc_u * sc_u + mu_u * mu_u, axis=1) - D)
             - jnp.sum(jnp.log(sc_u), axis=1))
    kle_i = (0.5 * (jnp.sum(sc_i * sc_i + mu_i * mu_i, axis=1) - D)
             - jnp.sum(jnp.log(sc_i), axis=1))
    kle_ref[...] = jnp.stack([kle_u, kle_i], axis=1)

    @pl.when(pl.program_id(0) == 0)
    def _():
        std_ref[...] = jnp.full((1, 1), jnp.sqrt(1.0 / jax.nn.softplus(alpha_ref[0])),
                                dtype=jnp.float32)


_BM = 2048


def _tc_compute(eu, ei, m2, r2, epsb2, epse2, alpha, gb):
    grid = (B // _BM,)
    return pl.pallas_call(
        _tc_body,
        grid=grid,
        in_specs=[
            pl.BlockSpec((_BM, 2 * D), lambda i: (i, 0)),
            pl.BlockSpec((_BM, 2 * D), lambda i: (i, 0)),
            pl.BlockSpec((_BM, 2), lambda i: (i, 0)),
            pl.BlockSpec((_BM, 2), lambda i: (i, 0)),
            pl.BlockSpec((_BM, 2), lambda i: (i, 0)),
            pl.BlockSpec((_BM, 2 * D), lambda i: (i, 0)),
            pl.BlockSpec(memory_space=pltpu.SMEM),
            pl.BlockSpec(memory_space=pltpu.SMEM),
        ],
        out_specs=[
            pl.BlockSpec((_BM, 1), lambda i: (i, 0)),
            pl.BlockSpec((_BM, 2), lambda i: (i, 0)),
            pl.BlockSpec((_BM, 2), lambda i: (i, 0)),
            pl.BlockSpec((1, 1), lambda i: (0, 0)),
        ],
        out_shape=[
            jax.ShapeDtypeStruct((B, 1), jnp.float32),
            jax.ShapeDtypeStruct((B, 2), jnp.float32),
            jax.ShapeDtypeStruct((B, 2), jnp.float32),
            jax.ShapeDtypeStruct((1, 1), jnp.float32),
        ],
    )(eu, ei, m2, r2, epsb2, epse2, alpha, gb)


def kernel(x, bias_table, entity_table, alpha, global_bias):
    idx = x.astype(jnp.int32).reshape(TWO_B)
    eu, ei = _sc_gather_entity(idx, entity_table)
    m_flat, r_flat = _sc_gather_bias(idx, bias_table[:, 0], bias_table[:, 1])
    eps_b = jax.random.normal(jax.random.key(1), (1, TWO_B),
                              dtype=jnp.float32).reshape(B, 2)
    eps_e = jax.random.normal(jax.random.key(2), (1, TWO_B, D),
                              dtype=jnp.float32).reshape(B, 2 * D)
    pred2, klb2, kle2, std2 = _tc_compute(eu, ei, m_flat.reshape(B, 2),
                                          r_flat.reshape(B, 2), eps_b, eps_e,
                                          alpha, global_bias)
    return (pred2.reshape(B), std2.reshape(1),
            klb2.reshape(TWO_B), kle2.reshape(TWO_B))
